# baseline (speedup 1.0000x reference)
import jax
import jax.numpy as jnp
from jax.experimental import pallas as pl
from jax.experimental.pallas import tpu as pltpu

NCLS = 80
NA = 6400
NG = 50
TOPK = 10
_BIG = 3.0e38


def _bce(x, t):
    return jnp.clip(x, 0.0) - x * t + jnp.log1p(jnp.exp(-jnp.abs(x)))


def _smooth_l1(d):
    ad = jnp.abs(d)
    return jnp.where(ad < 1.0, 0.5 * ad * ad, ad - 0.5)


def _loss_kernel(predt_ref, gtb_ref, oh_ref, box_ref, obj_ref, cls_ref):
    predt = predt_ref[0]
    px1 = predt[0:1, :]
    py1 = predt[1:2, :]
    px2 = predt[2:3, :]
    py2 = predt[3:4, :]
    obj = predt[4:5, :]
    xt = predt[5:5 + NCLS, :]

    gtb = gtb_ref[0]
    gx1 = gtb[:, 0:1]
    gy1 = gtb[:, 1:2]
    gx2 = gtb[:, 2:3]
    gy2 = gtb[:, 3:4]

    tlx = jnp.maximum(gx1, px1)
    tly = jnp.maximum(gy1, py1)
    brx = jnp.minimum(gx2, px2)
    bry = jnp.minimum(gy2, py2)
    inter = jnp.clip(brx - tlx, 0.0) * jnp.clip(bry - tly, 0.0)
    area_g = (gx2 - gx1) * (gy2 - gy1)
    area_p = (px2 - px1) * (py2 - py1)
    pair_ious = jnp.clip(inter / (area_g + area_p - inter + 1e-8), 0.0, 1.0)

    p = jax.nn.sigmoid(xt)
    logp = jnp.log(p + 1e-8)
    log1mp = jnp.log(1.0 - p + 1e-8)
    oh = oh_ref[0]
    c1 = jnp.dot(oh, logp, preferred_element_type=jnp.float32)
    c2 = jnp.dot(1.0 - oh, log1mp, preferred_element_type=jnp.float32)
    cost = -(c1 + c2) - 3.0 * jnp.log(pair_ious + 1e-8)

    acc = jnp.zeros((NG, 1), jnp.float32)
    vals = pair_ious
    for _ in range(TOPK):
        m = jnp.max(vals, axis=1, keepdims=True)
        acc = acc + jnp.maximum(m, 0.0)
        vals = jnp.where(vals == m, -1.0, vals)
    kf = jnp.clip(jnp.floor(acc), 1.0, float(TOPK))

    thr = jnp.full((NG, 1), _BIG, jnp.float32)
    work = cost
    for i in range(TOPK):
        m = jnp.min(work, axis=1, keepdims=True)
        thr = jnp.where(float(i) < kf, m, thr)
        work = jnp.where(work == m, _BIG, work)
    matched = (cost <= thr).astype(jnp.float32)

    cnt_n = jnp.sum(matched, axis=0, keepdims=True)
    iota_g = jax.lax.broadcasted_iota(jnp.int32, (NG, NA), 0)
    cmin = jnp.min(cost, axis=0, keepdims=True)
    gfirst = jnp.min(jnp.where(cost == cmin, iota_g, NG), axis=0, keepdims=True)
    best = (iota_g == gfirst).astype(jnp.float32)
    M = jnp.where(cnt_n > 1.0, best, matched)
    mask = (cnt_n > 0.0).astype(jnp.float32)
    num_fg = jnp.maximum(jnp.sum(mask), 1.0)

    ax1 = jnp.sum(M * gx1, axis=0, keepdims=True)
    ay1 = jnp.sum(M * gy1, axis=0, keepdims=True)
    ax2 = jnp.sum(M * gx2, axis=0, keepdims=True)
    ay2 = jnp.sum(M * gy2, axis=0, keepdims=True)

    sl = (_smooth_l1(px1 - ax1) + _smooth_l1(py1 - ay1)
          + _smooth_l1(px2 - ax2) + _smooth_l1(py2 - ay2))
    box_l = jnp.sum(mask * sl) / (num_fg * 4.0)

    itlx = jnp.maximum(px1, ax1)
    itly = jnp.maximum(py1, ay1)
    ibrx = jnp.minimum(px2, ax2)
    ibry = jnp.minimum(py2, ay2)
    inter2 = jnp.clip(ibrx - itlx, 0.0) * jnp.clip(ibry - itly, 0.0)
    area_a = (ax2 - ax1) * (ay2 - ay1)
    iou_t = inter2 / (area_p + area_a - inter2 + 1e-8)
    s = jax.nn.sigmoid(obj)
    obj_l = jnp.sum(mask * _bce(s, iou_t)) / num_fg

    sp_sum = -jnp.sum(log1mp, axis=0, keepdims=True)
    t_oh = jax.lax.dot_general(oh, M, (((0,), (0,)), ((), ())),
                               preferred_element_type=jnp.float32)
    xl = jnp.sum(xt * t_oh, axis=0, keepdims=True)
    cls_l = jnp.sum(mask * (sp_sum - xl)) / (num_fg * float(NCLS))

    box_ref[0] = jnp.broadcast_to(box_l, (1, 1))
    obj_ref[0] = jnp.broadcast_to(obj_l, (1, 1))
    cls_ref[0] = jnp.broadcast_to(cls_l, (1, 1))


def kernel(preds, gt_boxes, gt_labels):
    B = preds.shape[0]
    preds_t = jnp.einsum(
        'cd,bnd->bcn', jnp.eye(5 + NCLS, dtype=jnp.float32),
        preds.reshape(B, preds.shape[1], NA, 5 + NCLS)[:, 0],
        preferred_element_type=jnp.float32,
        precision=jax.lax.Precision.HIGHEST)
    oh = jax.nn.one_hot(gt_labels, NCLS, dtype=jnp.float32)

    box_a, obj_a, cls_a = pl.pallas_call(
        _loss_kernel,
        grid=(B,),
        in_specs=[
            pl.BlockSpec((1, 5 + NCLS, NA), lambda b: (b, 0, 0)),
            pl.BlockSpec((1, NG, 4), lambda b: (b, 0, 0)),
            pl.BlockSpec((1, NG, NCLS), lambda b: (b, 0, 0)),
        ],
        out_specs=[
            pl.BlockSpec((1, 1, 1), lambda b: (b, 0, 0)),
            pl.BlockSpec((1, 1, 1), lambda b: (b, 0, 0)),
            pl.BlockSpec((1, 1, 1), lambda b: (b, 0, 0)),
        ],
        out_shape=[jax.ShapeDtypeStruct((B, 1, 1), jnp.float32)] * 3,
        compiler_params=pltpu.CompilerParams(
            dimension_semantics=("parallel",)),
    )(preds_t, gt_boxes, oh)

    box = jnp.sum(box_a)
    obj = jnp.sum(obj_a)
    cls = jnp.sum(cls_a)
    bf = float(B)
    total = (5.0 * box + 1.0 * obj + 1.0 * cls) / bf
    return (total, box / bf, obj / bf, cls / bf)


# speedup vs baseline: 1.3432x; 1.3432x over previous
import jax
import jax.numpy as jnp
from jax.experimental import pallas as pl
from jax.experimental.pallas import tpu as pltpu

NCLS = 80
NA = 6400
NG = 50
TOPK = 10
_BIG = 3.0e38


def _bce(x, t):
    return jnp.clip(x, 0.0) - x * t + jnp.log1p(jnp.exp(-jnp.abs(x)))


def _smooth_l1(d):
    ad = jnp.abs(d)
    return jnp.where(ad < 1.0, 0.5 * ad * ad, ad - 0.5)


def _loss_kernel(pred_ref, gtb_ref, oh_ref, box_ref, obj_ref, cls_ref):
    predt = jnp.transpose(pred_ref[0, 0], (1, 0))
    px1 = predt[0:1, :]
    py1 = predt[1:2, :]
    px2 = predt[2:3, :]
    py2 = predt[3:4, :]
    obj = predt[4:5, :]
    xt = predt[5:5 + NCLS, :]

    gtb = gtb_ref[0]
    gx1 = gtb[:, 0:1]
    gy1 = gtb[:, 1:2]
    gx2 = gtb[:, 2:3]
    gy2 = gtb[:, 3:4]

    tlx = jnp.maximum(gx1, px1)
    tly = jnp.maximum(gy1, py1)
    brx = jnp.minimum(gx2, px2)
    bry = jnp.minimum(gy2, py2)
    inter = jnp.clip(brx - tlx, 0.0) * jnp.clip(bry - tly, 0.0)
    area_g = (gx2 - gx1) * (gy2 - gy1)
    area_p = (px2 - px1) * (py2 - py1)
    pair_ious = jnp.clip(inter / (area_g + area_p - inter + 1e-8), 0.0, 1.0)

    p = jax.nn.sigmoid(xt)
    logp = jnp.log(p + 1e-8)
    log1mp = jnp.log(1.0 - p + 1e-8)
    oh = oh_ref[0]
    c1 = jnp.dot(oh, logp, preferred_element_type=jnp.float32)
    c2 = jnp.dot(1.0 - oh, log1mp, preferred_element_type=jnp.float32)
    cost = -(c1 + c2) - 3.0 * jnp.log(pair_ious + 1e-8)

    acc = jnp.zeros((NG, 1), jnp.float32)
    vals = pair_ious
    for _ in range(TOPK):
        m = jnp.max(vals, axis=1, keepdims=True)
        acc = acc + jnp.maximum(m, 0.0)
        vals = jnp.where(vals == m, -1.0, vals)
    kf = jnp.clip(jnp.floor(acc), 1.0, float(TOPK))

    thr = jnp.full((NG, 1), _BIG, jnp.float32)
    work = cost
    for i in range(TOPK):
        m = jnp.min(work, axis=1, keepdims=True)
        thr = jnp.where(float(i) < kf, m, thr)
        work = jnp.where(work == m, _BIG, work)
    matched = (cost <= thr).astype(jnp.float32)

    cnt_n = jnp.sum(matched, axis=0, keepdims=True)
    iota_g = jax.lax.broadcasted_iota(jnp.int32, (NG, NA), 0)
    cmin = jnp.min(cost, axis=0, keepdims=True)
    gfirst = jnp.min(jnp.where(cost == cmin, iota_g, NG), axis=0, keepdims=True)
    best = (iota_g == gfirst).astype(jnp.float32)
    M = jnp.where(cnt_n > 1.0, best, matched)
    mask = (cnt_n > 0.0).astype(jnp.float32)
    num_fg = jnp.maximum(jnp.sum(mask), 1.0)

    ax1 = jnp.sum(M * gx1, axis=0, keepdims=True)
    ay1 = jnp.sum(M * gy1, axis=0, keepdims=True)
    ax2 = jnp.sum(M * gx2, axis=0, keepdims=True)
    ay2 = jnp.sum(M * gy2, axis=0, keepdims=True)

    sl = (_smooth_l1(px1 - ax1) + _smooth_l1(py1 - ay1)
          + _smooth_l1(px2 - ax2) + _smooth_l1(py2 - ay2))
    box_l = jnp.sum(mask * sl) / (num_fg * 4.0)

    itlx = jnp.maximum(px1, ax1)
    itly = jnp.maximum(py1, ay1)
    ibrx = jnp.minimum(px2, ax2)
    ibry = jnp.minimum(py2, ay2)
    inter2 = jnp.clip(ibrx - itlx, 0.0) * jnp.clip(ibry - itly, 0.0)
    area_a = (ax2 - ax1) * (ay2 - ay1)
    iou_t = inter2 / (area_p + area_a - inter2 + 1e-8)
    s = jax.nn.sigmoid(obj)
    obj_l = jnp.sum(mask * _bce(s, iou_t)) / num_fg

    sp_sum = -jnp.sum(log1mp, axis=0, keepdims=True)
    t_oh = jax.lax.dot_general(oh, M, (((0,), (0,)), ((), ())),
                               preferred_element_type=jnp.float32)
    xl = jnp.sum(xt * t_oh, axis=0, keepdims=True)
    cls_l = jnp.sum(mask * (sp_sum - xl)) / (num_fg * float(NCLS))

    box_ref[0] = jnp.broadcast_to(box_l, (1, 1))
    obj_ref[0] = jnp.broadcast_to(obj_l, (1, 1))
    cls_ref[0] = jnp.broadcast_to(cls_l, (1, 1))


def kernel(preds, gt_boxes, gt_labels):
    B = preds.shape[0]
    preds_r = preds.reshape(B, preds.shape[1], NA, 5 + NCLS)
    oh = jax.nn.one_hot(gt_labels, NCLS, dtype=jnp.float32)

    box_a, obj_a, cls_a = pl.pallas_call(
        _loss_kernel,
        grid=(B,),
        in_specs=[
            pl.BlockSpec((1, 1, NA, 5 + NCLS), lambda b: (b, 0, 0, 0)),
            pl.BlockSpec((1, NG, 4), lambda b: (b, 0, 0)),
            pl.BlockSpec((1, NG, NCLS), lambda b: (b, 0, 0)),
        ],
        out_specs=[
            pl.BlockSpec((1, 1, 1), lambda b: (b, 0, 0)),
            pl.BlockSpec((1, 1, 1), lambda b: (b, 0, 0)),
            pl.BlockSpec((1, 1, 1), lambda b: (b, 0, 0)),
        ],
        out_shape=[jax.ShapeDtypeStruct((B, 1, 1), jnp.float32)] * 3,
        compiler_params=pltpu.CompilerParams(
            dimension_semantics=("parallel",)),
    )(preds_r, gt_boxes, oh)

    box = jnp.sum(box_a)
    obj = jnp.sum(obj_a)
    cls = jnp.sum(cls_a)
    bf = float(B)
    total = (5.0 * box + 1.0 * obj + 1.0 * cls) / bf
    return (total, box / bf, obj / bf, cls / bf)
